# baseline (speedup 1.0000x reference)
import functools

import jax
import jax.numpy as jnp
from jax import lax
from jax.experimental import pallas as pl
from jax.experimental.pallas import tpu as pltpu
from jax.experimental.pallas import tpu_sc as plsc

NC = 2
NS = 16
NW = NC * NS
CHUNK = 128


def kernel(task_idx, table):
    (B,) = task_idx.shape
    V, D = table.shape
    b_per_w = B // NW
    K = b_per_w // CHUNK

    idx3 = task_idx.reshape(NW, K, CHUNK).astype(jnp.int32)
    mesh = plsc.VectorSubcoreMesh(core_axis_name="c", subcore_axis_name="s")

    @functools.partial(
        pl.kernel,
        mesh=mesh,
        out_type=jax.ShapeDtypeStruct((NW, K, CHUNK, D), jnp.float32),
        scratch_types=[
            pltpu.VMEM((K, CHUNK), jnp.int32),
            pltpu.VMEM((K, CHUNK, D), jnp.float32),
            pltpu.SemaphoreType.DMA,
        ],
        compiler_params=pltpu.CompilerParams(use_tc_tiling_on_sc=False),
    )
    def gather_kernel(idx_hbm, table_hbm, out_hbm, idx_v, rows_v, sem):
        wid = lax.axis_index("s") * NC + lax.axis_index("c")
        pltpu.sync_copy(idx_hbm.at[wid], idx_v)
        copies = [
            pltpu.async_copy(table_hbm.at[idx_v.at[j]], rows_v.at[j], sem)
            for j in range(K)
        ]
        for c in copies:
            c.wait()
        pltpu.sync_copy(rows_v, out_hbm.at[wid])

    out = gather_kernel(idx3, table)
    return out.reshape(B, D)


# speedup vs baseline: 3.8322x; 3.8322x over previous
import functools

import jax
import jax.numpy as jnp
from jax import lax
from jax.experimental import pallas as pl
from jax.experimental.pallas import tpu as pltpu
from jax.experimental.pallas import tpu_sc as plsc

NC = 2
NS = 16
NW = NC * NS
G = 16


def kernel(task_idx, table):
    (B,) = task_idx.shape
    V, D = table.shape
    b_per_w = B // NW
    n_groups = b_per_w // G

    tableT = table.T
    mesh = plsc.VectorSubcoreMesh(core_axis_name="c", subcore_axis_name="s")

    @functools.partial(
        pl.kernel,
        mesh=mesh,
        out_type=jax.ShapeDtypeStruct((D, B), jnp.float32),
        scratch_types=[
            pltpu.VMEM((b_per_w,), jnp.int32),
            pltpu.VMEM((G, D, 128), jnp.float32),
            pltpu.VMEM((D, b_per_w), jnp.float32),
            pltpu.SemaphoreType.DMA,
        ],
        compiler_params=pltpu.CompilerParams(
            use_tc_tiling_on_sc=True, needs_layout_passes=False
        ),
    )
    def gather_kernel(idx_hbm, tableT_hbm, outT_hbm, idx_v, blk_v, cols_v, sem):
        wid = lax.axis_index("s") * NC + lax.axis_index("c")
        base = wid * b_per_w
        pltpu.sync_copy(idx_hbm.at[pl.ds(base, b_per_w)], idx_v)

        jot = lax.iota(jnp.int32, 16)

        @pl.loop(0, n_groups)
        def per_group(g):
            k0 = g * G
            ivec = idx_v[pl.ds(k0, 16)]
            copies = []
            for j in range(G):
                i = ivec[j]
                cc = pl.multiple_of((i >> 7) * 128, 128)
                for r in range(D // 8):
                    copies.append(
                        pltpu.async_copy(
                            tableT_hbm.at[pl.ds(8 * r, 8), pl.ds(cc, 128)],
                            blk_v.at[j, pl.ds(8 * r, 8)],
                            sem,
                        )
                    )
            for c in copies:
                c.wait()
            lanes = ivec & 127
            for d in range(D):
                dv = jnp.full((16,), d, jnp.int32)
                v = plsc.load_gather(blk_v, [jot, dv, lanes])
                cols_v[d, pl.ds(k0, 16)] = v

        pltpu.sync_copy(cols_v, outT_hbm.at[:, pl.ds(base, b_per_w)])

    outT = gather_kernel(task_idx, tableT)
    return outT.T


# speedup vs baseline: 3.8917x; 1.0155x over previous
import functools

import jax
import jax.numpy as jnp
from jax import lax
from jax.experimental import pallas as pl
from jax.experimental.pallas import tpu as pltpu
from jax.experimental.pallas import tpu_sc as plsc

NC = 2
NS = 16
NW = NC * NS
G = 16


def kernel(task_idx, table):
    (B,) = task_idx.shape
    V, D = table.shape
    b_per_w = B // NW
    n_groups = b_per_w // G

    tableT = table.T
    mesh = plsc.VectorSubcoreMesh(core_axis_name="c", subcore_axis_name="s")

    @functools.partial(
        pl.kernel,
        mesh=mesh,
        out_type=jax.ShapeDtypeStruct((D, B), jnp.float32),
        scratch_types=[
            pltpu.VMEM((b_per_w,), jnp.int32),
            pltpu.VMEM((G, D, 128), jnp.float32),
            pltpu.VMEM((D, b_per_w), jnp.float32),
            pltpu.SemaphoreType.DMA,
        ],
        compiler_params=pltpu.CompilerParams(
            use_tc_tiling_on_sc=True, needs_layout_passes=False
        ),
    )
    def gather_kernel(idx_hbm, tableT_hbm, outT_hbm, idx_v, blk_v, cols_v, sem):
        wid = lax.axis_index("s") * NC + lax.axis_index("c")
        base = wid * b_per_w
        pltpu.sync_copy(idx_hbm.at[pl.ds(base, b_per_w)], idx_v)

        jot = lax.iota(jnp.int32, 16)

        @pl.loop(0, n_groups)
        def per_group(g):
            k0 = g * G
            ivec = idx_v[pl.ds(k0, 16)]
            copies = []
            for j in range(G):
                i = ivec[j]
                cc = pl.multiple_of((i >> 7) * 128, 128)
                copies.append(
                    pltpu.async_copy(
                        tableT_hbm.at[:, pl.ds(cc, 128)], blk_v.at[j], sem
                    )
                )
            for c in copies:
                c.wait()
            lanes = ivec & 127
            for d in range(D):
                dv = jnp.full((16,), d, jnp.int32)
                v = plsc.load_gather(blk_v, [jot, dv, lanes])
                cols_v[d, pl.ds(k0, 16)] = v

        pltpu.sync_copy(cols_v, outT_hbm.at[:, pl.ds(base, b_per_w)])

    outT = gather_kernel(task_idx, tableT)
    return outT.T
